# baseline (speedup 1.0000x reference)
import functools

import jax
import jax.numpy as jnp
from jax import lax
from jax.experimental import pallas as pl
from jax.experimental.pallas import tpu as pltpu
from jax.experimental.pallas import tpu_sc as plsc

B, S, H = 4, 4096, 2048
N = B * S
D1, D2 = 512, 128
CUT0, CUT1 = 25000, 50000
SZ1, SZ2 = 25000, 50000

NW = 32
TOK_PER_W = N // NW
LANES = 16
CK = 16
NCH = TOK_PER_W // CK

_SC_PARAMS = pltpu.CompilerParams(needs_layout_passes=False)


def _compact(ids_v, lo, hi, dst_idx_ref, dst_pos_ref, base, sub):
    lane = lax.iota(jnp.int32, LANES)
    count = jnp.int32(0)
    for i in range(TOK_PER_W // LANES):
        v = ids_v[pl.ds(i * LANES, LANES)]
        m = (v >= lo) & (v < hi) if lo > 0 else v < hi
        mi = jnp.where(m, jnp.int32(1), jnp.int32(0))
        offs = count + plsc.cumsum(mi) - mi
        if dst_idx_ref is not None:
            plsc.store_scatter(dst_idx_ref, [offs], v - sub, mask=m)
        if dst_pos_ref is not None:
            pos = base + i * LANES + lane
            plsc.store_scatter(dst_pos_ref, [offs >> 4, offs & 15], pos, mask=m)
        count = count + jnp.sum(mi)
    return count


def _pad_dup0(count, idx_ref, pos_ref):
    lane = lax.iota(jnp.int32, LANES)
    padded = ((count + CK - 1) // CK) * CK

    @pl.when(count > 0)
    def _():
        zeros = jnp.zeros((LANES,), jnp.int32)
        for j in range(2):
            idx = count + j * LANES + lane
            mj = idx < padded
            if idx_ref is not None:
                i0 = plsc.load_gather(idx_ref, [zeros])
                plsc.store_scatter(idx_ref, [idx], i0, mask=mj)
            if pos_ref is not None:
                p0 = plsc.load_gather(pos_ref, [zeros, zeros])
                plsc.store_scatter(pos_ref, [idx >> 4, idx & 15], p0, mask=mj)

    return padded // CK


@functools.lru_cache(maxsize=1)
def _make_sc_gather12c():
    mesh = plsc.VectorSubcoreMesh(core_axis_name="c", subcore_axis_name="s")

    @functools.partial(
        pl.kernel,
        mesh=mesh,
        compiler_params=_SC_PARAMS,
        out_type=(
            jax.ShapeDtypeStruct((N, D1), jnp.float32),
            jax.ShapeDtypeStruct((N, D2), jnp.float32),
            jax.ShapeDtypeStruct((NW, LANES), jnp.int32),
            jax.ShapeDtypeStruct((N, H), jnp.float32),
        ),
        scratch_types=[
            pltpu.VMEM((TOK_PER_W,), jnp.int32),
            pltpu.VMEM((TOK_PER_W,), jnp.int32),
            pltpu.VMEM((TOK_PER_W,), jnp.int32),
            pltpu.VMEM((LANES,), jnp.int32),
            pltpu.VMEM((2, CK, D1), jnp.float32),
            pltpu.VMEM((2, CK, D2), jnp.float32),
            pltpu.SemaphoreType.DMA,
            pltpu.SemaphoreType.DMA,
        ],
    )
    def _sc_gather12c(ids_hbm, emb1_hbm, emb2_hbm,
                      g1c_hbm, g2c_hbm, cnt_hbm, _out_stage,
                      ids_v, cidx1_v, cidx2_v, cnt_v, buf1, buf2, sem1, sem2):
        wid = lax.axis_index("s") * 2 + lax.axis_index("c")
        base = wid * TOK_PER_W
        pltpu.sync_copy(ids_hbm.at[pl.ds(base, TOK_PER_W)], ids_v)
        c1 = _compact(ids_v, CUT0, CUT1, cidx1_v, None, base, CUT0)
        c2 = _compact(ids_v, CUT1, 1 << 30, cidx2_v, None, base, CUT1)
        lane = lax.iota(jnp.int32, LANES)
        cnt_v[...] = jnp.where(lane == 0, c1, jnp.where(lane == 1, c2, 0))
        pltpu.sync_copy(cnt_v, cnt_hbm.at[wid])
        nch1 = _pad_dup0(c1, cidx1_v, None)
        nch2 = _pad_dup0(c2, cidx2_v, None)

        def body1(c, prev):
            b = c % 2
            cp = pltpu.async_copy(
                emb1_hbm.at[cidx1_v.at[pl.ds(c * CK, CK)]], buf1.at[b], sem1)
            cp.wait()
            pltpu.async_copy(
                buf1.at[b], g1c_hbm.at[pl.ds(base + c * CK, CK)], sem1).wait()
            return c

        def body2(c, prev):
            b = c % 2
            cp = pltpu.async_copy(
                emb2_hbm.at[cidx2_v.at[pl.ds(c * CK, CK)]], buf2.at[b], sem2)
            cp.wait()
            pltpu.async_copy(
                buf2.at[b], g2c_hbm.at[pl.ds(base + c * CK, CK)], sem2).wait()
            return c

        lax.fori_loop(0, nch1, body1, 0)
        lax.fori_loop(0, nch2, body2, 0)

    return _sc_gather12c


@functools.lru_cache(maxsize=1)
def _make_sc_scatter_emb0():
    mesh = plsc.VectorSubcoreMesh(core_axis_name="c", subcore_axis_name="s")

    @functools.partial(
        pl.kernel,
        mesh=mesh,
        compiler_params=_SC_PARAMS,
        out_type=(),
        scratch_types=[
            pltpu.VMEM((TOK_PER_W,), jnp.int32),
            pltpu.VMEM((NCH, CK), jnp.int32),
            pltpu.VMEM((TOK_PER_W,), jnp.int32),
            pltpu.VMEM((2, CK, H), jnp.float32),
            pltpu.SemaphoreType.DMA,
            pltpu.SemaphoreType.DMA,
        ],
    )
    def _sc_scatter_emb0(ids_hbm, emb0_hbm, out_ref,
                         ids_v, pos0_v, cidx0_v, buf, gsem, ssem):
        wid = lax.axis_index("s") * 2 + lax.axis_index("c")
        base = wid * TOK_PER_W
        pltpu.sync_copy(ids_hbm.at[pl.ds(base, TOK_PER_W)], ids_v)
        c0 = _compact(ids_v, 0, CUT0, cidx0_v, pos0_v, base, 0)
        nch0 = _pad_dup0(c0, cidx0_v, pos0_v)

        @pl.when(nch0 > 0)
        def _():
            pltpu.async_copy(
                emb0_hbm.at[cidx0_v.at[pl.ds(0, CK)]], buf.at[0], gsem).wait()

        def body0(c, _):
            b = c & 1
            cp = pltpu.async_copy(buf.at[b], out_ref.at[pos0_v.at[c]], ssem)

            @pl.when(c + 1 < nch0)
            def _():
                pltpu.async_copy(
                    emb0_hbm.at[cidx0_v.at[pl.ds((c + 1) * CK, CK)]],
                    buf.at[1 - b], gsem).wait()

            cp.wait()
            return 0

        lax.fori_loop(0, nch0, body0, 0)

    return _sc_scatter_emb0


@functools.lru_cache(maxsize=1)
def _make_sc_scatter_proj():
    mesh = plsc.VectorSubcoreMesh(core_axis_name="c", subcore_axis_name="s")

    @functools.partial(
        pl.kernel,
        mesh=mesh,
        compiler_params=_SC_PARAMS,
        out_type=(),
        scratch_types=[
            pltpu.VMEM((TOK_PER_W,), jnp.int32),
            pltpu.VMEM((NCH, CK), jnp.int32),
            pltpu.VMEM((NCH, CK), jnp.int32),
            pltpu.VMEM((2, CK, H), jnp.float32),
            pltpu.SemaphoreType.DMA,
            pltpu.SemaphoreType.DMA,
        ],
    )
    def _sc_scatter_proj(ids_hbm, p1c_hbm, p2c_hbm, out_ref,
                         ids_v, pos1_v, pos2_v, buf, gsem, ssem):
        wid = lax.axis_index("s") * 2 + lax.axis_index("c")
        base = wid * TOK_PER_W
        pltpu.sync_copy(ids_hbm.at[pl.ds(base, TOK_PER_W)], ids_v)
        c1 = _compact(ids_v, CUT0, CUT1, None, pos1_v, base, 0)
        c2 = _compact(ids_v, CUT1, 1 << 30, None, pos2_v, base, 0)
        nch1 = _pad_dup0(c1, None, pos1_v)
        nch2 = _pad_dup0(c2, None, pos2_v)

        def run(pc_hbm, pos_ref, nch):
            @pl.when(nch > 0)
            def _():
                pltpu.async_copy(
                    pc_hbm.at[pl.ds(base, CK)], buf.at[0], gsem).wait()

            def body(c, _):
                b = c & 1
                cp = pltpu.async_copy(buf.at[b], out_ref.at[pos_ref.at[c]], ssem)

                @pl.when(c + 1 < nch)
                def _():
                    pltpu.async_copy(
                        pc_hbm.at[pl.ds(base + (c + 1) * CK, CK)],
                        buf.at[1 - b], gsem).wait()

                cp.wait()
                return 0

            lax.fori_loop(0, nch, body, 0)

        run(p1c_hbm, pos1_v, nch1)
        run(p2c_hbm, pos2_v, nch2)

    return _sc_scatter_proj


TB = 256
HB = TOK_PER_W // TB


def _clamp_blk(w, h, cnt, col):
    nblk = jnp.maximum((cnt[w, col] + TB - 1) // TB, 1)
    return w * HB + jnp.minimum(h, nblk - 1)


def _tc_body(cnt_ref, g1_ref, g2_ref, w1_ref, b1_ref, w2_ref, b2_ref,
             p1_ref, p2_ref):
    w = pl.program_id(0)
    h = pl.program_id(1)

    @pl.when(h * TB < cnt_ref[w, 0])
    def _():
        p1_ref[...] = lax.dot_general(
            g1_ref[...].astype(jnp.bfloat16), w1_ref[...].astype(jnp.bfloat16),
            (((1,), (1,)), ((), ())),
            preferred_element_type=jnp.float32) + b1_ref[...]

    @pl.when(h * TB < cnt_ref[w, 1])
    def _():
        p2_ref[...] = lax.dot_general(
            g2_ref[...].astype(jnp.bfloat16), w2_ref[...].astype(jnp.bfloat16),
            (((1,), (1,)), ((), ())),
            preferred_element_type=jnp.float32) + b2_ref[...]


_tc_project = pl.pallas_call(
    _tc_body,
    grid_spec=pltpu.PrefetchScalarGridSpec(
        num_scalar_prefetch=1,
        grid=(NW, HB),
        in_specs=[
            pl.BlockSpec((TB, D1), lambda w, h, cnt: (_clamp_blk(w, h, cnt, 0), 0)),
            pl.BlockSpec((TB, D2), lambda w, h, cnt: (_clamp_blk(w, h, cnt, 1), 0)),
            pl.BlockSpec((H, D1), lambda w, h, cnt: (0, 0)),
            pl.BlockSpec((1, H), lambda w, h, cnt: (0, 0)),
            pl.BlockSpec((H, D2), lambda w, h, cnt: (0, 0)),
            pl.BlockSpec((1, H), lambda w, h, cnt: (0, 0)),
        ],
        out_specs=[
            pl.BlockSpec((TB, H), lambda w, h, cnt: (_clamp_blk(w, h, cnt, 0), 0)),
            pl.BlockSpec((TB, H), lambda w, h, cnt: (_clamp_blk(w, h, cnt, 1), 0)),
        ],
    ),
    out_shape=[
        jax.ShapeDtypeStruct((N, H), jnp.float32),
        jax.ShapeDtypeStruct((N, H), jnp.float32),
    ],
)


def kernel(input_ids, emb0, emb1, emb2, proj1_w, proj1_b, proj2_w, proj2_b):
    ids = input_ids.reshape(-1).astype(jnp.int32)
    g1c, g2c, cnt, out_stage = _make_sc_gather12c()(ids, emb1, emb2)
    out_ref = jax.new_ref(out_stage)
    _make_sc_scatter_emb0()(ids, emb0, out_ref)
    p1c, p2c = _tc_project(cnt, g1c, g2c,
                           proj1_w, proj1_b.reshape(1, H),
                           proj2_w, proj2_b.reshape(1, H))
    _make_sc_scatter_proj()(ids, p1c, p2c, out_ref)
    return out_ref[...].reshape(B, S, H)


# speedup vs baseline: 1.0017x; 1.0017x over previous
import functools

import jax
import jax.numpy as jnp
from jax import lax
from jax.experimental import pallas as pl
from jax.experimental.pallas import tpu as pltpu
from jax.experimental.pallas import tpu_sc as plsc

B, S, H = 4, 4096, 2048
N = B * S
D1, D2 = 512, 128
CUT0, CUT1 = 25000, 50000
SZ1, SZ2 = 25000, 50000

NW = 32
TOK_PER_W = N // NW
LANES = 16
CK = 16
NCH = TOK_PER_W // CK

_SC_PARAMS = pltpu.CompilerParams(needs_layout_passes=False)


def _compact(ids_v, lo, hi, dst_idx_ref, dst_pos_ref, base, sub):
    lane = lax.iota(jnp.int32, LANES)
    count = jnp.int32(0)
    for i in range(TOK_PER_W // LANES):
        v = ids_v[pl.ds(i * LANES, LANES)]
        m = (v >= lo) & (v < hi) if lo > 0 else v < hi
        mi = jnp.where(m, jnp.int32(1), jnp.int32(0))
        offs = count + plsc.cumsum(mi) - mi
        if dst_idx_ref is not None:
            plsc.store_scatter(dst_idx_ref, [offs], v - sub, mask=m)
        if dst_pos_ref is not None:
            pos = base + i * LANES + lane
            plsc.store_scatter(dst_pos_ref, [offs >> 4, offs & 15], pos, mask=m)
        count = count + jnp.sum(mi)
    return count


def _pad_dup0(count, idx_ref, pos_ref):
    lane = lax.iota(jnp.int32, LANES)
    padded = ((count + CK - 1) // CK) * CK

    @pl.when(count > 0)
    def _():
        zeros = jnp.zeros((LANES,), jnp.int32)
        for j in range(2):
            idx = count + j * LANES + lane
            mj = idx < padded
            if idx_ref is not None:
                i0 = plsc.load_gather(idx_ref, [zeros])
                plsc.store_scatter(idx_ref, [idx], i0, mask=mj)
            if pos_ref is not None:
                p0 = plsc.load_gather(pos_ref, [zeros, zeros])
                plsc.store_scatter(pos_ref, [idx >> 4, idx & 15], p0, mask=mj)

    return padded // CK


@functools.lru_cache(maxsize=1)
def _make_sc_gather12c():
    mesh = plsc.VectorSubcoreMesh(core_axis_name="c", subcore_axis_name="s")

    @functools.partial(
        pl.kernel,
        mesh=mesh,
        compiler_params=_SC_PARAMS,
        out_type=(
            jax.ShapeDtypeStruct((N, D1), jnp.float32),
            jax.ShapeDtypeStruct((N, D2), jnp.float32),
            jax.ShapeDtypeStruct((NW, LANES), jnp.int32),
            jax.ShapeDtypeStruct((N, H), jnp.float32),
        ),
        scratch_types=[
            pltpu.VMEM((TOK_PER_W,), jnp.int32),
            pltpu.VMEM((TOK_PER_W,), jnp.int32),
            pltpu.VMEM((TOK_PER_W,), jnp.int32),
            pltpu.VMEM((LANES,), jnp.int32),
            pltpu.VMEM((2, CK, D1), jnp.float32),
            pltpu.VMEM((2, CK, D2), jnp.float32),
            pltpu.SemaphoreType.DMA,
            pltpu.SemaphoreType.DMA,
        ],
    )
    def _sc_gather12c(ids_hbm, emb1_hbm, emb2_hbm,
                      g1c_hbm, g2c_hbm, cnt_hbm, _out_stage,
                      ids_v, cidx1_v, cidx2_v, cnt_v, buf1, buf2, sem1, sem2):
        wid = lax.axis_index("s") * 2 + lax.axis_index("c")
        base = wid * TOK_PER_W
        pltpu.sync_copy(ids_hbm.at[pl.ds(base, TOK_PER_W)], ids_v)
        c1 = _compact(ids_v, CUT0, CUT1, cidx1_v, None, base, CUT0)
        c2 = _compact(ids_v, CUT1, 1 << 30, cidx2_v, None, base, CUT1)
        lane = lax.iota(jnp.int32, LANES)
        cnt_v[...] = jnp.where(lane == 0, c1, jnp.where(lane == 1, c2, 0))
        pltpu.sync_copy(cnt_v, cnt_hbm.at[wid])
        nch1 = _pad_dup0(c1, cidx1_v, None)
        nch2 = _pad_dup0(c2, cidx2_v, None)

        def body1(c, prev):
            b = c % 2
            cp = pltpu.async_copy(
                emb1_hbm.at[cidx1_v.at[pl.ds(c * CK, CK)]], buf1.at[b], sem1)
            cp.wait()
            pltpu.async_copy(
                buf1.at[b], g1c_hbm.at[pl.ds(base + c * CK, CK)], sem1).wait()
            return c

        def body2(c, prev):
            b = c % 2
            cp = pltpu.async_copy(
                emb2_hbm.at[cidx2_v.at[pl.ds(c * CK, CK)]], buf2.at[b], sem2)
            cp.wait()
            pltpu.async_copy(
                buf2.at[b], g2c_hbm.at[pl.ds(base + c * CK, CK)], sem2).wait()
            return c

        lax.fori_loop(0, nch1, body1, 0)
        lax.fori_loop(0, nch2, body2, 0)

    return _sc_gather12c


@functools.lru_cache(maxsize=1)
def _make_sc_scatter_emb0():
    mesh = plsc.VectorSubcoreMesh(core_axis_name="c", subcore_axis_name="s")

    @functools.partial(
        pl.kernel,
        mesh=mesh,
        compiler_params=_SC_PARAMS,
        out_type=(),
        scratch_types=[
            pltpu.VMEM((TOK_PER_W,), jnp.int32),
            pltpu.VMEM((NCH, CK), jnp.int32),
            pltpu.VMEM((TOK_PER_W,), jnp.int32),
            pltpu.VMEM((2, CK, H), jnp.float32),
            pltpu.SemaphoreType.DMA,
            pltpu.SemaphoreType.DMA,
        ],
    )
    def _sc_scatter_emb0(ids_hbm, emb0_hbm, out_ref,
                         ids_v, pos0_v, cidx0_v, buf, gsem, ssem):
        wid = lax.axis_index("s") * 2 + lax.axis_index("c")
        base = wid * TOK_PER_W
        pltpu.sync_copy(ids_hbm.at[pl.ds(base, TOK_PER_W)], ids_v)
        c0 = _compact(ids_v, 0, CUT0, cidx0_v, pos0_v, base, 0)
        nch0 = _pad_dup0(c0, cidx0_v, pos0_v)

        @pl.when(nch0 > 0)
        def _():
            pltpu.async_copy(
                emb0_hbm.at[cidx0_v.at[pl.ds(0, CK)]], buf.at[0], gsem).wait()

        def body0(c, _):
            b = c & 1
            cp = pltpu.async_copy(buf.at[b], out_ref.at[pos0_v.at[c]], ssem)

            @pl.when(c + 1 < nch0)
            def _():
                pltpu.async_copy(
                    emb0_hbm.at[cidx0_v.at[pl.ds((c + 1) * CK, CK)]],
                    buf.at[1 - b], gsem).wait()

            cp.wait()
            return 0

        lax.fori_loop(0, nch0, body0, 0)

    return _sc_scatter_emb0


@functools.lru_cache(maxsize=1)
def _make_sc_scatter_proj():
    mesh = plsc.VectorSubcoreMesh(core_axis_name="c", subcore_axis_name="s")

    @functools.partial(
        pl.kernel,
        mesh=mesh,
        compiler_params=_SC_PARAMS,
        out_type=(),
        scratch_types=[
            pltpu.VMEM((TOK_PER_W,), jnp.int32),
            pltpu.VMEM((NCH, CK), jnp.int32),
            pltpu.VMEM((NCH, CK), jnp.int32),
            pltpu.VMEM((2, CK, H), jnp.float32),
            pltpu.SemaphoreType.DMA,
            pltpu.SemaphoreType.DMA,
        ],
    )
    def _sc_scatter_proj(ids_hbm, p1c_hbm, p2c_hbm, out_ref,
                         ids_v, pos1_v, pos2_v, buf, gsem, ssem):
        wid = lax.axis_index("s") * 2 + lax.axis_index("c")
        base = wid * TOK_PER_W
        pltpu.sync_copy(ids_hbm.at[pl.ds(base, TOK_PER_W)], ids_v)
        c1 = _compact(ids_v, CUT0, CUT1, None, pos1_v, base, 0)
        c2 = _compact(ids_v, CUT1, 1 << 30, None, pos2_v, base, 0)
        nch1 = _pad_dup0(c1, None, pos1_v)
        nch2 = _pad_dup0(c2, None, pos2_v)

        def run(pc_hbm, pos_ref, nch):
            @pl.when(nch > 0)
            def _():
                pltpu.async_copy(
                    pc_hbm.at[pl.ds(base, CK)], buf.at[0], gsem).wait()

            def body(c, _):
                b = c & 1
                cp = pltpu.async_copy(buf.at[b], out_ref.at[pos_ref.at[c]], ssem)

                @pl.when(c + 1 < nch)
                def _():
                    pltpu.async_copy(
                        pc_hbm.at[pl.ds(base + (c + 1) * CK, CK)],
                        buf.at[1 - b], gsem).wait()

                cp.wait()
                return 0

            lax.fori_loop(0, nch, body, 0)

        run(p1c_hbm, pos1_v, nch1)
        run(p2c_hbm, pos2_v, nch2)

    return _sc_scatter_proj


TB = 256
HB = TOK_PER_W // TB


def _clamp_blk(w, h, cnt, col):
    nblk = jnp.maximum((cnt[w, col] + TB - 1) // TB, 1)
    return w * HB + jnp.minimum(h, nblk - 1)


def _tc_body(cnt_ref, g1_ref, g2_ref, w1_ref, b1_ref, w2_ref, b2_ref,
             p1_ref, p2_ref):
    w = pl.program_id(0)
    h = pl.program_id(1)

    @pl.when(h * TB < cnt_ref[w, 0])
    def _():
        p1_ref[...] = lax.dot_general(
            g1_ref[...], w1_ref[...], (((1,), (1,)), ((), ())),
            preferred_element_type=jnp.float32) + b1_ref[...]

    @pl.when(h * TB < cnt_ref[w, 1])
    def _():
        p2_ref[...] = lax.dot_general(
            g2_ref[...], w2_ref[...], (((1,), (1,)), ((), ())),
            preferred_element_type=jnp.float32) + b2_ref[...]


_tc_project = pl.pallas_call(
    _tc_body,
    grid_spec=pltpu.PrefetchScalarGridSpec(
        num_scalar_prefetch=1,
        grid=(NW, HB),
        in_specs=[
            pl.BlockSpec((TB, D1), lambda w, h, cnt: (_clamp_blk(w, h, cnt, 0), 0)),
            pl.BlockSpec((TB, D2), lambda w, h, cnt: (_clamp_blk(w, h, cnt, 1), 0)),
            pl.BlockSpec((H, D1), lambda w, h, cnt: (0, 0)),
            pl.BlockSpec((1, H), lambda w, h, cnt: (0, 0)),
            pl.BlockSpec((H, D2), lambda w, h, cnt: (0, 0)),
            pl.BlockSpec((1, H), lambda w, h, cnt: (0, 0)),
        ],
        out_specs=[
            pl.BlockSpec((TB, H), lambda w, h, cnt: (_clamp_blk(w, h, cnt, 0), 0)),
            pl.BlockSpec((TB, H), lambda w, h, cnt: (_clamp_blk(w, h, cnt, 1), 0)),
        ],
    ),
    out_shape=[
        jax.ShapeDtypeStruct((N, H), jnp.float32),
        jax.ShapeDtypeStruct((N, H), jnp.float32),
    ],
)


def kernel(input_ids, emb0, emb1, emb2, proj1_w, proj1_b, proj2_w, proj2_b):
    ids = input_ids.reshape(-1).astype(jnp.int32)
    g1c, g2c, cnt, out_stage = _make_sc_gather12c()(ids, emb1, emb2)
    out_ref = jax.new_ref(out_stage)
    _make_sc_scatter_emb0()(ids, emb0, out_ref)
    p1c, p2c = _tc_project(cnt, g1c, g2c,
                           proj1_w, proj1_b.reshape(1, H),
                           proj2_w, proj2_b.reshape(1, H))
    _make_sc_scatter_proj()(ids, p1c, p2c, out_ref)
    return out_ref[...].reshape(B, S, H)
